# Initial kernel scaffold; baseline (speedup 1.0000x reference)
import jax
import jax.numpy as jnp
from jax.experimental import pallas as pl


def kernel(species, in_features, atom_index12, distances, total_charges, W1, b1, W2, b2, nW1, nb1, nW2, nb2, fW1, fb1, fW2, fb2, decay_prefactor, decay_factor):
    raise NotImplementedError("write your pallas kernel here")


# speedup vs baseline: 13.7475x; 13.7475x over previous
import functools

import jax
import jax.numpy as jnp
from jax import lax
from jax.experimental import pallas as pl
from jax.experimental.pallas import tpu as pltpu
from jax.experimental.pallas import tpu_sc as plsc

B, A, D = 32, 256, 128
N = B * A
P = 131072
E = 4
M0, H0 = 256, 256
NB, HN = 128, 128
F, HF = 384, 128
CUTOFF = 5.2

BLK = 1024
PCOLS = P // N


def _celu(x):
    return jnp.where(x > 0, x, jnp.exp(x) - 1.0)


def _mlp_select(x, W1_ref, b1_ref, W2_ref, b2_ref, oh):
    out = None
    for e in range(E):
        h = jnp.dot(x, W1_ref[e], preferred_element_type=jnp.float32)
        h = _celu(h + b1_ref[e])
        o = jnp.dot(h, W2_ref[e], preferred_element_type=jnp.float32) + b2_ref[e]
        o = o * oh[:, e:e + 1]
        out = o if out is None else out + o
    return out


def _a_body(x_ref, oh_ref, dist_ref, sc_ref,
            W1_ref, b1_ref, W2_ref, b2_ref,
            nW1_ref, nb1_ref, nW2_ref, nb2_ref,
            int_ref, ngh_ref, dec_ref):
    x = x_ref[...]
    oh = oh_ref[...]
    internal = _mlp_select(x, W1_ref, b1_ref, W2_ref, b2_ref, oh)
    int_ref[...] = internal
    ngh_ref[...] = _mlp_select(internal, nW1_ref, nb1_ref, nW2_ref, nb2_ref, oh)
    dist = dist_ref[...]
    pf2 = sc_ref[0]
    df2 = sc_ref[1]
    c = 0.5 * jnp.cos((jnp.pi / CUTOFF) * dist) + 0.5
    cut = jnp.where(dist < CUTOFF, c, 0.0)
    dec_ref[...] = pf2 * jnp.exp(-df2 * dist) * cut


def _phase_a(feats, oh, dist2d, scal, W1, b1, W2, b2, nW1, nb1, nW2, nb2):
    grid = (N // BLK,)
    const = lambda shape: pl.BlockSpec(shape, lambda i: (0,) * len(shape))
    return pl.pallas_call(
        _a_body,
        grid=grid,
        in_specs=[
            pl.BlockSpec((BLK, D), lambda i: (i, 0)),
            pl.BlockSpec((BLK, E), lambda i: (i, 0)),
            pl.BlockSpec((BLK, PCOLS), lambda i: (i, 0)),
            pl.BlockSpec(memory_space=pltpu.SMEM),
            const((E, D, H0)), const((E, H0)),
            const((E, H0, M0)), const((E, M0)),
            const((E, M0, HN)), const((E, HN)),
            const((E, HN, NB)), const((E, NB)),
        ],
        out_specs=[
            pl.BlockSpec((BLK, M0), lambda i: (i, 0)),
            pl.BlockSpec((BLK, NB), lambda i: (i, 0)),
            pl.BlockSpec((BLK, PCOLS), lambda i: (i, 0)),
        ],
        out_shape=[
            jax.ShapeDtypeStruct((N, M0), jnp.float32),
            jax.ShapeDtypeStruct((N, NB), jnp.float32),
            jax.ShapeDtypeStruct((N, PCOLS), jnp.float32),
        ],
    )(feats, oh, dist2d, scal, W1, b1, W2, b2, nW1, nb1, nW2, nb2)


NC = 2
NS = 16
NW = NC * NS
CH = 128
PAIRS_PER_W = P // NW
NCHUNK = PAIRS_PER_W // CH
ROWS_PER_TILE = N // NS


def _sc_scale(rows, dv):
    def body(g, _):
        dvec = dv[pl.ds(g * 16, 16)]
        base = g * 16
        for c in range(16):
            d = dvec[c]
            for j in range(8):
                s = pl.ds(j * 16, 16)
                rows[base + c, s] = rows[base + c, s] * d
        return 0
    lax.fori_loop(0, CH // 16, body, 0)


def _sc_body(idx0_h, idx1_h, dec_h, neigh_h, out_h, ia, ib, dv, rows, acc, sem):
    cid = lax.axis_index("c")
    sid = lax.axis_index("s")
    wid = cid * NS + sid

    zero16 = jnp.zeros((16,), jnp.float32)

    def zb(i, _):
        for j in range(8):
            rows[i, pl.ds(j * 16, 16)] = zero16
        return 0
    lax.fori_loop(0, CH, zb, 0)
    for t in range(ROWS_PER_TILE // CH):
        pltpu.sync_copy(rows, acc.at[pl.ds(sid * ROWS_PER_TILE + t * CH, CH)])
    plsc.subcore_barrier()

    def chunk(k, _):
        base = wid * PAIRS_PER_W + k * CH
        pltpu.sync_copy(idx0_h.at[pl.ds(base, CH)], ia)
        pltpu.sync_copy(idx1_h.at[pl.ds(base, CH)], ib)
        pltpu.sync_copy(dec_h.at[pl.ds(base, CH)], dv)
        pltpu.async_copy(neigh_h.at[ib], rows, sem).wait()
        _sc_scale(rows, dv)
        pltpu.sync_copy(rows, acc.at[ia], add=True)
        pltpu.async_copy(neigh_h.at[ia], rows, sem).wait()
        _sc_scale(rows, dv)
        pltpu.sync_copy(rows, acc.at[ib], add=True)
        return 0
    lax.fori_loop(0, NCHUNK, chunk, 0)
    plsc.subcore_barrier()

    for t in range(ROWS_PER_TILE // CH):
        r0 = sid * ROWS_PER_TILE + t * CH
        pltpu.sync_copy(acc.at[pl.ds(r0, CH)], out_h.at[cid, pl.ds(r0, CH)])


@functools.partial(
    pl.kernel,
    mesh=plsc.VectorSubcoreMesh(core_axis_name="c", subcore_axis_name="s"),
    out_type=jax.ShapeDtypeStruct((NC, N, NB), jnp.float32),
    scratch_types=[
        pltpu.VMEM((CH,), jnp.int32),
        pltpu.VMEM((CH,), jnp.int32),
        pltpu.VMEM((CH,), jnp.float32),
        pltpu.VMEM((CH, NB), jnp.float32),
        pltpu.VMEM_SHARED((N, NB), jnp.float32),
        pltpu.SemaphoreType.DMA,
    ],
)
def _sc_scatter(idx0_h, idx1_h, dec_h, neigh_h, out_h, ia, ib, dv, rows, acc, sem):
    _sc_body(idx0_h, idx1_h, dec_h, neigh_h, out_h, ia, ib, dv, rows, acc, sem)


def _c_body(int_ref, mg_ref, oh_ref, fW1a_ref, fW1b_ref, fb1_ref, fW2_ref,
            fb2_ref, pre_ref):
    internal = int_ref[...]
    merged = mg_ref[0] + mg_ref[1]
    oh = oh_ref[...]
    out = None
    for e in range(E):
        h = jnp.dot(internal, fW1a_ref[e], preferred_element_type=jnp.float32)
        h = h + jnp.dot(merged, fW1b_ref[e], preferred_element_type=jnp.float32)
        h = _celu(h + fb1_ref[e])
        pc = jnp.sum(h * fW2_ref[e][None, :], axis=1, keepdims=True) + fb2_ref[e]
        pc = pc * oh[:, e:e + 1]
        out = pc if out is None else out + pc
    pre_ref[...] = out


def _phase_c(internal, merged2, oh, fW1a, fW1b, fb1, fW2r, fb2r):
    grid = (N // BLK,)
    const = lambda shape: pl.BlockSpec(shape, lambda i: (0,) * len(shape))
    return pl.pallas_call(
        _c_body,
        grid=grid,
        in_specs=[
            pl.BlockSpec((BLK, M0), lambda i: (i, 0)),
            pl.BlockSpec((NC, BLK, NB), lambda i: (0, i, 0)),
            pl.BlockSpec((BLK, E), lambda i: (i, 0)),
            const((E, M0, HF)), const((E, NB, HF)), const((E, HF)),
            const((E, HF)),
            pl.BlockSpec(memory_space=pltpu.SMEM),
        ],
        out_specs=pl.BlockSpec((BLK, 1), lambda i: (i, 0)),
        out_shape=jax.ShapeDtypeStruct((N, 1), jnp.float32),
    )(internal, merged2, oh, fW1a, fW1b, fb1, fW2r, fb2r)


def _d_body(pc_ref, sp_ref, tc_ref, ch_ref):
    pc = pc_ref[...]
    mask = sp_ref[...] != -1
    tp = jnp.sum(pc, axis=1, keepdims=True)
    cnt = jnp.sum(mask.astype(jnp.float32), axis=1, keepdims=True)
    ch = pc + (tc_ref[...] - tp) / cnt
    ch_ref[...] = jnp.where(mask, ch, 0.0)


def _phase_d(precharges, species, total_charges):
    return pl.pallas_call(
        _d_body,
        out_shape=jax.ShapeDtypeStruct((B, A), jnp.float32),
    )(precharges, species, total_charges)


def kernel(species, in_features, atom_index12, distances, total_charges,
           W1, b1, W2, b2, nW1, nb1, nW2, nb2, fW1, fb1, fW2, fb2,
           decay_prefactor, decay_factor):
    sp = species.reshape(-1).astype(jnp.int32)
    feats = in_features.reshape(-1, D)
    oh = (sp[:, None] == jnp.arange(E, dtype=jnp.int32)[None, :]).astype(jnp.float32)
    scal = jnp.stack([decay_prefactor.astype(jnp.float32) ** 2,
                      decay_factor.astype(jnp.float32) ** 2])
    dist2d = distances.reshape(N, PCOLS)

    internal, neigh, dec2d = _phase_a(
        feats, oh, dist2d, scal, W1, b1, W2, b2, nW1, nb1, nW2, nb2)

    decay = dec2d.reshape(-1)
    idx0 = atom_index12[0].astype(jnp.int32)
    idx1 = atom_index12[1].astype(jnp.int32)
    merged2 = _sc_scatter(idx0, idx1, decay, neigh)

    fW1a = fW1[:, :M0, :]
    fW1b = fW1[:, M0:, :]
    fW2r = fW2.reshape(E, HF)
    fb2r = fb2.reshape(E)
    pre = _phase_c(internal, merged2, oh, fW1a, fW1b, fb1, fW2r, fb2r)

    precharges = pre.reshape(B, A)
    charges = _phase_d(precharges, species, total_charges.reshape(B, 1))
    return species, charges, precharges
